# baseline (speedup 1.0000x reference)
import jax
import jax.numpy as jnp
from jax.experimental import pallas as pl

B = 8; IMG = 224; P = 16; HID = 768; HEADS = 12; DKV = 64; DFF = 1024
E = 16; DEPTH = 4; NCLS = 1000
G = IMG // P
S = G * G + 1
T = B * S
CAP = int(2.0 * T / E)
CAPP = 200
TRASH = E * CAPP

_INTERPRET = False


def _dot(a, b):
    return jax.lax.dot_general(a, b, (((a.ndim - 1,), (0,)), ((), ())))


def _ln(x, s, b):
    m = x.mean(-1, keepdims=True)
    v = ((x - m) ** 2).mean(-1, keepdims=True)
    return (x - m) / jnp.sqrt(v + 1e-6) * s + b


def _mm_body(x_ref, w_ref, b_ref, o_ref):
    o_ref[0] = _dot(x_ref[0], w_ref[...]) + b_ref[0]


def _scores_body(q_ref, k_ref, o_ref):
    o_ref[0, 0] = jax.lax.dot_general(
        q_ref[0, 0], k_ref[0, 0], (((1,), (1,)), ((), ()))) / jnp.sqrt(
            jnp.float32(DKV))


def _attnv_body(p_ref, v_ref, o_ref):
    o_ref[0, 0] = _dot(p_ref[0, 0], v_ref[0, 0])


def _proj_body(o_ref_in, w_ref, b_ref, h_ref, out_ref):
    out_ref[0] = h_ref[0] + (_dot(o_ref_in[0], w_ref[...]) + b_ref[0])


def _gate_body(x_ref, w_ref, o_ref):
    o_ref[...] = _dot(x_ref[...], w_ref[...])


def _route_body(pr_ref, scat1_ref, scat2_ref, g1_ref, g2_ref):
    probs = pr_ref[...]
    iota = jax.lax.broadcasted_iota(jnp.int32, (T, E), 1)

    def top(pr):
        mx = jnp.max(pr, -1, keepdims=True)
        idx = jnp.min(jnp.where(pr == mx, iota, E), -1, keepdims=True)
        m = (iota == idx).astype(jnp.float32)
        return idx, m

    idx1, m1 = top(probs)
    idx2, m2 = top(probs * (1.0 - m1))

    tri = (jax.lax.broadcasted_iota(jnp.int32, (S, S), 0)
           >= jax.lax.broadcasted_iota(jnp.int32, (S, S), 1)).astype(
               jnp.float32)

    def chunked_cumsum(m):
        off = jnp.zeros((1, E), jnp.float32)
        outs = []
        for c in range(B):
            blk = m[c * S:(c + 1) * S]
            outs.append(_dot(tri, blk) + off)
            off = off + jnp.sum(blk, 0, keepdims=True)
        return jnp.concatenate(outs, 0), off

    cs1, sum1 = chunked_cumsum(m1)
    cs2, _ = chunked_cumsum(m2)
    loc1 = cs1 - m1
    loc2 = cs2 - m2 + sum1
    m1m = m1 * (loc1 < CAP)
    m2m = m2 * (loc2 < CAP)
    p1 = jnp.sum(loc1 * m1m, -1, keepdims=True)
    p2 = jnp.sum(loc2 * m2m, -1, keepdims=True)
    g1 = jnp.sum(probs * m1m, -1, keepdims=True)
    g2 = jnp.sum(probs * m2m, -1, keepdims=True)
    den = g1 + g2 + 1e-9
    g1_ref[...] = g1 / den
    g2_ref[...] = g2 / den
    v1 = jnp.sum(m1m, -1, keepdims=True) > 0.5
    v2 = jnp.sum(m2m, -1, keepdims=True) > 0.5
    scat1_ref[...] = jnp.where(v1, idx1 * CAPP + p1.astype(jnp.int32), TRASH)
    scat2_ref[...] = jnp.where(v2, idx2 * CAPP + p2.astype(jnp.int32), TRASH)


def _ffn_body(s1_ref, s2_ref, hn_ref, w1_ref, b1_ref, w2_ref, b2_ref, eo_ref):
    e = pl.program_id(0)
    rows = jax.lax.broadcasted_iota(jnp.int32, (CAPP, T), 0) + e * CAPP
    s1 = s1_ref[0]
    s2 = s2_ref[0]
    a = ((rows == s1[None, :]).astype(jnp.float32)
         + (rows == s2[None, :]).astype(jnp.float32))
    ein = _dot(a, hn_ref[...])
    h1 = jax.nn.gelu(_dot(ein, w1_ref[0]) + b1_ref[0, 0])
    eo_ref[0] = _dot(h1, w2_ref[0]) + b2_ref[0, 0]


def _comb_body(s1_ref, s2_ref, g1_ref, g2_ref, h_ref, eo_ref, out_ref):
    iot = jax.lax.broadcasted_iota(jnp.int32, (S, E * CAPP), 1)
    w = (g1_ref[0] * (iot == s1_ref[0]).astype(jnp.float32)
         + g2_ref[0] * (iot == s2_ref[0]).astype(jnp.float32))
    out_ref[0] = h_ref[0] + _dot(w, eo_ref[...])


def _final_body(h_ref, s_ref, b_ref, o_ref):
    hn = _ln(h_ref[0], s_ref[0], b_ref[0])
    o_ref[0] = jnp.mean(hn, 0, keepdims=True)


def _cls_body(p_ref, w_ref, b_ref, o_ref):
    o_ref[...] = _dot(p_ref[...], w_ref[...]) + b_ref[0]


def _full(shape):
    n = len(shape)
    return pl.BlockSpec(shape, lambda *_: (0,) * n)


def _per_b(shape):
    n = len(shape)
    return pl.BlockSpec((1,) + shape[1:], lambda b, *_: (b,) + (0,) * (n - 1))


def _mm3(x, w, bias):
    n, r, _ = x.shape
    return pl.pallas_call(
        _mm_body,
        grid=(n,),
        in_specs=[_per_b(x.shape), _full(w.shape), _full((1, w.shape[1]))],
        out_specs=_per_b((n, r, w.shape[1])),
        out_shape=jax.ShapeDtypeStruct((n, r, w.shape[1]), jnp.float32),
        interpret=_INTERPRET,
    )(x, w, bias.reshape(1, -1))


def _scores(q, k):
    spec = pl.BlockSpec((1, 1, S, DKV), lambda b, h: (b, h, 0, 0))
    ospec = pl.BlockSpec((1, 1, S, S), lambda b, h: (b, h, 0, 0))
    return pl.pallas_call(
        _scores_body,
        grid=(B, HEADS),
        in_specs=[spec, spec],
        out_specs=ospec,
        out_shape=jax.ShapeDtypeStruct((B, HEADS, S, S), jnp.float32),
        interpret=_INTERPRET,
    )(q, k)


def _attnv(p, v):
    pspec = pl.BlockSpec((1, 1, S, S), lambda b, h: (b, h, 0, 0))
    vspec = pl.BlockSpec((1, 1, S, DKV), lambda b, h: (b, h, 0, 0))
    return pl.pallas_call(
        _attnv_body,
        grid=(B, HEADS),
        in_specs=[pspec, vspec],
        out_specs=vspec,
        out_shape=jax.ShapeDtypeStruct((B, HEADS, S, DKV), jnp.float32),
        interpret=_INTERPRET,
    )(p, v)


def _proj(o, Wo, bo, h):
    return pl.pallas_call(
        _proj_body,
        grid=(B,),
        in_specs=[_per_b(o.shape), _full(Wo.shape), _full((1, HID)),
                  _per_b(h.shape)],
        out_specs=_per_b((B, S, HID)),
        out_shape=jax.ShapeDtypeStruct((B, S, HID), jnp.float32),
        interpret=_INTERPRET,
    )(o, Wo, bo.reshape(1, HID), h)


def _gate(hn2, gate_w):
    return pl.pallas_call(
        _gate_body,
        out_shape=jax.ShapeDtypeStruct((T, E), jnp.float32),
        interpret=_INTERPRET,
    )(hn2, gate_w)


def _route(probs):
    return pl.pallas_call(
        _route_body,
        out_shape=[jax.ShapeDtypeStruct((T, 1), jnp.int32),
                   jax.ShapeDtypeStruct((T, 1), jnp.int32),
                   jax.ShapeDtypeStruct((T, 1), jnp.float32),
                   jax.ShapeDtypeStruct((T, 1), jnp.float32)],
        interpret=_INTERPRET,
    )(probs)


def _ffn(s1r, s2r, hn, W1, b1, W2, b2):
    return pl.pallas_call(
        _ffn_body,
        grid=(E,),
        in_specs=[_full((1, T)), _full((1, T)), _full((T, HID)),
                  pl.BlockSpec((1, HID, DFF), lambda e: (e, 0, 0)),
                  pl.BlockSpec((1, 1, DFF), lambda e: (e, 0, 0)),
                  pl.BlockSpec((1, DFF, HID), lambda e: (e, 0, 0)),
                  pl.BlockSpec((1, 1, HID), lambda e: (e, 0, 0))],
        out_specs=pl.BlockSpec((1, CAPP, HID), lambda e: (e, 0, 0)),
        out_shape=jax.ShapeDtypeStruct((E, CAPP, HID), jnp.float32),
        interpret=_INTERPRET,
    )(s1r, s2r, hn, W1, b1.reshape(E, 1, DFF), W2, b2.reshape(E, 1, HID))


def _combine(s1, s2, g1, g2, h, eo):
    return pl.pallas_call(
        _comb_body,
        grid=(B,),
        in_specs=[_per_b((B, S, 1)), _per_b((B, S, 1)),
                  _per_b((B, S, 1)), _per_b((B, S, 1)),
                  _per_b((B, S, HID)), _full((E * CAPP, HID))],
        out_specs=_per_b((B, S, HID)),
        out_shape=jax.ShapeDtypeStruct((B, S, HID), jnp.float32),
        interpret=_INTERPRET,
    )(s1, s2, g1, g2, h, eo)


def _final(h, s, b):
    return pl.pallas_call(
        _final_body,
        grid=(B,),
        in_specs=[_per_b(h.shape), _full((1, HID)), _full((1, HID))],
        out_specs=_per_b((B, 1, HID)),
        out_shape=jax.ShapeDtypeStruct((B, 1, HID), jnp.float32),
        interpret=_INTERPRET,
    )(h, s.reshape(1, HID), b.reshape(1, HID))


def _classify(pooled, Wc, bc):
    return pl.pallas_call(
        _cls_body,
        out_shape=jax.ShapeDtypeStruct((B, NCLS), jnp.float32),
        interpret=_INTERPRET,
    )(pooled, Wc, bc.reshape(1, NCLS))


def kernel(x, Wpatch, bpatch, cls_tok, pos_emb, ln1_s, ln1_b, ln2_s, ln2_b,
           Wq, bq, Wk, bk, Wv, bv, Wo, bo, gate_w, W1, b1, W2, b2,
           lnf_s, lnf_b, Wc, bc):
    xp = x.reshape(B, 3, G, P, G, P).transpose(0, 2, 4, 1, 3, 5).reshape(
        B, G * G, 3 * P * P)
    pe = _mm3(xp, Wpatch, bpatch)
    h = jnp.concatenate([jnp.broadcast_to(cls_tok, (B, 1, HID)), pe],
                        1) + pos_emb
    Wqkv = jnp.concatenate([Wq, Wk, Wv], 1)
    bqkv = jnp.concatenate([bq, bk, bv])
    for i in range(DEPTH):
        hn = _ln(h, ln1_s[i], ln1_b[i])
        qkv = _mm3(hn, Wqkv, bqkv)
        qh = qkv[:, :, 0:HID].reshape(B, S, HEADS, DKV).transpose(0, 2, 1, 3)
        kh = qkv[:, :, HID:2 * HID].reshape(B, S, HEADS, DKV).transpose(
            0, 2, 1, 3)
        vh = qkv[:, :, 2 * HID:].reshape(B, S, HEADS, DKV).transpose(
            0, 2, 1, 3)
        a = jax.nn.softmax(_scores(qh, kh), -1)
        o = _attnv(a, vh).transpose(0, 2, 1, 3).reshape(B, S, HID)
        h = _proj(o, Wo, bo, h)
        hn2 = _ln(h, ln2_s[i], ln2_b[i]).reshape(T, HID)
        probs = jax.nn.softmax(_gate(hn2, gate_w), -1)
        scat1, scat2, g1, g2 = _route(probs)
        eo = _ffn(scat1.T, scat2.T, hn2, W1, b1, W2, b2)
        h = _combine(scat1.reshape(B, S, 1), scat2.reshape(B, S, 1),
                     g1.reshape(B, S, 1), g2.reshape(B, S, 1), h,
                     eo.reshape(E * CAPP, HID))
    pooled = _final(h, lnf_s, lnf_b)
    return _classify(pooled.reshape(B, HID), Wc, bc)


# speedup vs baseline: 1.8848x; 1.8848x over previous
import jax
import jax.numpy as jnp
from jax.experimental import pallas as pl

B = 8; IMG = 224; P = 16; HID = 768; HEADS = 12; DKV = 64; DFF = 1024
E = 16; DEPTH = 4; NCLS = 1000
G = IMG // P
S = G * G + 1
T = B * S
CAP = int(2.0 * T / E)
CAPP = 200
TRASH = E * CAPP

_INTERPRET = False


def _dot(a, b):
    return jax.lax.dot_general(a, b, (((a.ndim - 1,), (0,)), ((), ())))


def _ln(x, s, b):
    m = x.mean(-1, keepdims=True)
    v = ((x - m) ** 2).mean(-1, keepdims=True)
    return (x - m) / jnp.sqrt(v + 1e-6) * s + b


def _mm_body(x_ref, w_ref, b_ref, o_ref):
    o_ref[0] = _dot(x_ref[0], w_ref[...]) + b_ref[0]


def _qkvs_body(hn_ref, w_ref, b_ref, qkv_ref, s_ref):
    qkv = _dot(hn_ref[0], w_ref[...]) + b_ref[0]
    qkv_ref[0] = qkv
    for hd in range(HEADS):
        q = qkv[:, hd * DKV:(hd + 1) * DKV]
        k = qkv[:, HID + hd * DKV:HID + (hd + 1) * DKV]
        s_ref[0, hd] = jax.lax.dot_general(
            q, k, (((1,), (1,)), ((), ()))) / jnp.sqrt(jnp.float32(DKV))


def _avproj_body(p_ref, qkv_ref, w_ref, b_ref, h_ref, out_ref):
    os = []
    for hd in range(HEADS):
        v = qkv_ref[0][:, 2 * HID + hd * DKV:2 * HID + (hd + 1) * DKV]
        os.append(_dot(p_ref[0, hd], v))
    o = jnp.concatenate(os, -1)
    out_ref[0] = h_ref[0] + (_dot(o, w_ref[...]) + b_ref[0])


def _gate_body(x_ref, w_ref, o_ref):
    o_ref[...] = _dot(x_ref[...], w_ref[...])


def _route_body(pr_ref, scat1_ref, scat2_ref, g1_ref, g2_ref):
    probs = pr_ref[...]
    iota = jax.lax.broadcasted_iota(jnp.int32, (T, E), 1)

    def top(pr):
        mx = jnp.max(pr, -1, keepdims=True)
        idx = jnp.min(jnp.where(pr == mx, iota, E), -1, keepdims=True)
        m = (iota == idx).astype(jnp.float32)
        return idx, m

    idx1, m1 = top(probs)
    idx2, m2 = top(probs * (1.0 - m1))

    tri = (jax.lax.broadcasted_iota(jnp.int32, (S, S), 0)
           >= jax.lax.broadcasted_iota(jnp.int32, (S, S), 1)).astype(
               jnp.float32)

    def chunked_cumsum(m):
        off = jnp.zeros((1, E), jnp.float32)
        outs = []
        for c in range(B):
            blk = m[c * S:(c + 1) * S]
            outs.append(_dot(tri, blk) + off)
            off = off + jnp.sum(blk, 0, keepdims=True)
        return jnp.concatenate(outs, 0), off

    cs1, sum1 = chunked_cumsum(m1)
    cs2, _ = chunked_cumsum(m2)
    loc1 = cs1 - m1
    loc2 = cs2 - m2 + sum1
    m1m = m1 * (loc1 < CAP)
    m2m = m2 * (loc2 < CAP)
    p1 = jnp.sum(loc1 * m1m, -1, keepdims=True)
    p2 = jnp.sum(loc2 * m2m, -1, keepdims=True)
    g1 = jnp.sum(probs * m1m, -1, keepdims=True)
    g2 = jnp.sum(probs * m2m, -1, keepdims=True)
    den = g1 + g2 + 1e-9
    g1_ref[...] = g1 / den
    g2_ref[...] = g2 / den
    v1 = jnp.sum(m1m, -1, keepdims=True) > 0.5
    v2 = jnp.sum(m2m, -1, keepdims=True) > 0.5
    scat1_ref[...] = jnp.where(v1, idx1 * CAPP + p1.astype(jnp.int32), TRASH)
    scat2_ref[...] = jnp.where(v2, idx2 * CAPP + p2.astype(jnp.int32), TRASH)


def _ffn_body(s1_ref, s2_ref, hn_ref, w1_ref, b1_ref, w2_ref, b2_ref, eo_ref):
    e = pl.program_id(0)
    rows = jax.lax.broadcasted_iota(jnp.int32, (CAPP, T), 0) + e * CAPP
    s1 = s1_ref[0]
    s2 = s2_ref[0]
    a = ((rows == s1[None, :]).astype(jnp.float32)
         + (rows == s2[None, :]).astype(jnp.float32))
    ein = _dot(a, hn_ref[...])
    h1 = jax.nn.gelu(_dot(ein, w1_ref[0]) + b1_ref[0, 0])
    eo_ref[0] = _dot(h1, w2_ref[0]) + b2_ref[0, 0]


def _comb_body(s1_ref, s2_ref, g1_ref, g2_ref, h_ref, eo_ref, out_ref):
    iot = jax.lax.broadcasted_iota(jnp.int32, (S, E * CAPP), 1)
    w = (g1_ref[0] * (iot == s1_ref[0]).astype(jnp.float32)
         + g2_ref[0] * (iot == s2_ref[0]).astype(jnp.float32))
    out_ref[0] = h_ref[0] + _dot(w, eo_ref[...])


def _final_body(h_ref, s_ref, b_ref, o_ref):
    hn = _ln(h_ref[0], s_ref[0], b_ref[0])
    o_ref[0] = jnp.mean(hn, 0, keepdims=True)


def _cls_body(p_ref, w_ref, b_ref, o_ref):
    o_ref[...] = _dot(p_ref[...], w_ref[...]) + b_ref[0]


def _full(shape):
    n = len(shape)
    return pl.BlockSpec(shape, lambda *_: (0,) * n)


def _per_b(shape):
    n = len(shape)
    return pl.BlockSpec((1,) + shape[1:], lambda b, *_: (b,) + (0,) * (n - 1))


def _mm3(x, w, bias):
    n, r, _ = x.shape
    return pl.pallas_call(
        _mm_body,
        grid=(n,),
        in_specs=[_per_b(x.shape), _full(w.shape), _full((1, w.shape[1]))],
        out_specs=_per_b((n, r, w.shape[1])),
        out_shape=jax.ShapeDtypeStruct((n, r, w.shape[1]), jnp.float32),
        interpret=_INTERPRET,
    )(x, w, bias.reshape(1, -1))


def _qkv_scores(hn, Wqkv, bqkv):
    return pl.pallas_call(
        _qkvs_body,
        grid=(B,),
        in_specs=[_per_b(hn.shape), _full(Wqkv.shape), _full((1, 3 * HID))],
        out_specs=[_per_b((B, S, 3 * HID)), _per_b((B, HEADS, S, S))],
        out_shape=[jax.ShapeDtypeStruct((B, S, 3 * HID), jnp.float32),
                   jax.ShapeDtypeStruct((B, HEADS, S, S), jnp.float32)],
        interpret=_INTERPRET,
    )(hn, Wqkv, bqkv.reshape(1, 3 * HID))


def _attnv_proj(p, qkv, Wo, bo, h):
    return pl.pallas_call(
        _avproj_body,
        grid=(B,),
        in_specs=[_per_b((B, HEADS, S, S)), _per_b((B, S, 3 * HID)),
                  _full(Wo.shape), _full((1, HID)), _per_b((B, S, HID))],
        out_specs=_per_b((B, S, HID)),
        out_shape=jax.ShapeDtypeStruct((B, S, HID), jnp.float32),
        interpret=_INTERPRET,
    )(p, qkv, Wo, bo.reshape(1, HID), h)


def _gate(hn2, gate_w):
    return pl.pallas_call(
        _gate_body,
        out_shape=jax.ShapeDtypeStruct((T, E), jnp.float32),
        interpret=_INTERPRET,
    )(hn2, gate_w)


def _route(probs):
    return pl.pallas_call(
        _route_body,
        out_shape=[jax.ShapeDtypeStruct((T, 1), jnp.int32),
                   jax.ShapeDtypeStruct((T, 1), jnp.int32),
                   jax.ShapeDtypeStruct((T, 1), jnp.float32),
                   jax.ShapeDtypeStruct((T, 1), jnp.float32)],
        interpret=_INTERPRET,
    )(probs)


def _ffn(s1r, s2r, hn, W1, b1, W2, b2):
    return pl.pallas_call(
        _ffn_body,
        grid=(E,),
        in_specs=[_full((1, T)), _full((1, T)), _full((T, HID)),
                  pl.BlockSpec((1, HID, DFF), lambda e: (e, 0, 0)),
                  pl.BlockSpec((1, 1, DFF), lambda e: (e, 0, 0)),
                  pl.BlockSpec((1, DFF, HID), lambda e: (e, 0, 0)),
                  pl.BlockSpec((1, 1, HID), lambda e: (e, 0, 0))],
        out_specs=pl.BlockSpec((1, CAPP, HID), lambda e: (e, 0, 0)),
        out_shape=jax.ShapeDtypeStruct((E, CAPP, HID), jnp.float32),
        interpret=_INTERPRET,
    )(s1r, s2r, hn, W1, b1.reshape(E, 1, DFF), W2, b2.reshape(E, 1, HID))


def _combine(s1, s2, g1, g2, h, eo):
    return pl.pallas_call(
        _comb_body,
        grid=(B,),
        in_specs=[_per_b((B, S, 1)), _per_b((B, S, 1)),
                  _per_b((B, S, 1)), _per_b((B, S, 1)),
                  _per_b((B, S, HID)), _full((E * CAPP, HID))],
        out_specs=_per_b((B, S, HID)),
        out_shape=jax.ShapeDtypeStruct((B, S, HID), jnp.float32),
        interpret=_INTERPRET,
    )(s1, s2, g1, g2, h, eo)


def _final(h, s, b):
    return pl.pallas_call(
        _final_body,
        grid=(B,),
        in_specs=[_per_b(h.shape), _full((1, HID)), _full((1, HID))],
        out_specs=_per_b((B, 1, HID)),
        out_shape=jax.ShapeDtypeStruct((B, 1, HID), jnp.float32),
        interpret=_INTERPRET,
    )(h, s.reshape(1, HID), b.reshape(1, HID))


def _classify(pooled, Wc, bc):
    return pl.pallas_call(
        _cls_body,
        out_shape=jax.ShapeDtypeStruct((B, NCLS), jnp.float32),
        interpret=_INTERPRET,
    )(pooled, Wc, bc.reshape(1, NCLS))


def kernel(x, Wpatch, bpatch, cls_tok, pos_emb, ln1_s, ln1_b, ln2_s, ln2_b,
           Wq, bq, Wk, bk, Wv, bv, Wo, bo, gate_w, W1, b1, W2, b2,
           lnf_s, lnf_b, Wc, bc):
    xp = x.reshape(B, 3, G, P, G, P).transpose(0, 2, 4, 1, 3, 5).reshape(
        B, G * G, 3 * P * P)
    pe = _mm3(xp, Wpatch, bpatch)
    h = jnp.concatenate([jnp.broadcast_to(cls_tok, (B, 1, HID)), pe],
                        1) + pos_emb
    Wqkv = jnp.concatenate([Wq, Wk, Wv], 1)
    bqkv = jnp.concatenate([bq, bk, bv])
    for i in range(DEPTH):
        hn = _ln(h, ln1_s[i], ln1_b[i])
        qkv, sc = _qkv_scores(hn, Wqkv, bqkv)
        a = jax.nn.softmax(sc, -1)
        h = _attnv_proj(a, qkv, Wo, bo, h)
        hn2 = _ln(h, ln2_s[i], ln2_b[i]).reshape(T, HID)
        probs = jax.nn.softmax(_gate(hn2, gate_w), -1)
        scat1, scat2, g1, g2 = _route(probs)
        eo = _ffn(scat1.T, scat2.T, hn2, W1, b1, W2, b2)
        h = _combine(scat1.reshape(B, S, 1), scat2.reshape(B, S, 1),
                     g1.reshape(B, S, 1), g2.reshape(B, S, 1), h,
                     eo.reshape(E * CAPP, HID))
    pooled = _final(h, lnf_s, lnf_b)
    return _classify(pooled.reshape(B, HID), Wc, bc)


# speedup vs baseline: 1.9153x; 1.0162x over previous
import jax
import jax.numpy as jnp
from jax.experimental import pallas as pl

B = 8; IMG = 224; P = 16; HID = 768; HEADS = 12; DKV = 64; DFF = 1024
E = 16; DEPTH = 4; NCLS = 1000
G = IMG // P
S = G * G + 1
T = B * S
CAP = int(2.0 * T / E)
CAPP = 200
TRASH = E * CAPP

_INTERPRET = False


def _dot(a, b):
    return jax.lax.dot_general(a, b, (((a.ndim - 1,), (0,)), ((), ())),
                               preferred_element_type=jnp.float32)


def _ln(x, s, b):
    m = x.mean(-1, keepdims=True)
    v = ((x - m) ** 2).mean(-1, keepdims=True)
    return (x - m) / jnp.sqrt(v + 1e-6) * s + b


def _mm_body(x_ref, w_ref, b_ref, o_ref):
    o_ref[0] = _dot(x_ref[0], w_ref[...]) + b_ref[0]


def _qkvs_body(hn_ref, w_ref, b_ref, v_ref, s_ref):
    qkv = _dot(hn_ref[0], w_ref[...]) + b_ref[0]
    v_ref[0] = qkv[:, 2 * HID:].astype(jnp.bfloat16)
    for hd in range(HEADS):
        q = qkv[:, hd * DKV:(hd + 1) * DKV]
        k = qkv[:, HID + hd * DKV:HID + (hd + 1) * DKV]
        s_ref[0, hd] = jax.lax.dot_general(
            q, k, (((1,), (1,)), ((), ()))) / jnp.sqrt(jnp.float32(DKV))


def _avproj_body(p_ref, v_ref, w_ref, b_ref, h_ref, out_ref):
    os = []
    for hd in range(HEADS):
        v = v_ref[0][:, hd * DKV:(hd + 1) * DKV]
        os.append(_dot(p_ref[0, hd], v))
    o = jnp.concatenate(os, -1)
    out_ref[0] = h_ref[0] + (_dot(o, w_ref[...]) + b_ref[0])


def _gate_body(x_ref, w_ref, o_ref):
    o_ref[...] = _dot(x_ref[...], w_ref[...])


def _route_body(pr_ref, scat1_ref, scat2_ref, g1_ref, g2_ref):
    probs = pr_ref[...]
    iota = jax.lax.broadcasted_iota(jnp.int32, (T, E), 1)

    def top(pr):
        mx = jnp.max(pr, -1, keepdims=True)
        idx = jnp.min(jnp.where(pr == mx, iota, E), -1, keepdims=True)
        m = (iota == idx).astype(jnp.float32)
        return idx, m

    idx1, m1 = top(probs)
    idx2, m2 = top(probs * (1.0 - m1))

    tri = (jax.lax.broadcasted_iota(jnp.int32, (S, S), 0)
           >= jax.lax.broadcasted_iota(jnp.int32, (S, S), 1)).astype(
               jnp.float32)

    def chunked_cumsum(m):
        off = jnp.zeros((1, E), jnp.float32)
        outs = []
        for c in range(B):
            blk = m[c * S:(c + 1) * S]
            outs.append(_dot(tri, blk) + off)
            off = off + jnp.sum(blk, 0, keepdims=True)
        return jnp.concatenate(outs, 0), off

    cs1, sum1 = chunked_cumsum(m1)
    cs2, _ = chunked_cumsum(m2)
    loc1 = cs1 - m1
    loc2 = cs2 - m2 + sum1
    m1m = m1 * (loc1 < CAP)
    m2m = m2 * (loc2 < CAP)
    p1 = jnp.sum(loc1 * m1m, -1, keepdims=True)
    p2 = jnp.sum(loc2 * m2m, -1, keepdims=True)
    g1 = jnp.sum(probs * m1m, -1, keepdims=True)
    g2 = jnp.sum(probs * m2m, -1, keepdims=True)
    den = g1 + g2 + 1e-9
    g1_ref[...] = g1 / den
    g2_ref[...] = g2 / den
    v1 = jnp.sum(m1m, -1, keepdims=True) > 0.5
    v2 = jnp.sum(m2m, -1, keepdims=True) > 0.5
    scat1_ref[...] = jnp.where(v1, idx1 * CAPP + p1.astype(jnp.int32), TRASH)
    scat2_ref[...] = jnp.where(v2, idx2 * CAPP + p2.astype(jnp.int32), TRASH)


def _ffn_body(s1_ref, s2_ref, hn_ref, w1_ref, b1_ref, w2_ref, b2_ref, eo_ref):
    e = pl.program_id(0)
    rows = jax.lax.broadcasted_iota(jnp.int32, (CAPP, T), 0) + e * CAPP
    s1 = s1_ref[0]
    s2 = s2_ref[0]
    a = ((rows == s1[None, :]).astype(jnp.float32)
         + (rows == s2[None, :]).astype(jnp.float32))
    ein = _dot(a, hn_ref[...])
    h1 = jax.nn.gelu(_dot(ein, w1_ref[0]) + b1_ref[0, 0])
    eo_ref[0] = (_dot(h1, w2_ref[0]) + b2_ref[0, 0]).astype(jnp.bfloat16)


def _comb_body(s1_ref, s2_ref, g1_ref, g2_ref, h_ref, eo_ref, out_ref):
    iot = jax.lax.broadcasted_iota(jnp.int32, (S, E * CAPP), 1)
    w = (g1_ref[0] * (iot == s1_ref[0]).astype(jnp.float32)
         + g2_ref[0] * (iot == s2_ref[0]).astype(jnp.float32))
    out_ref[0] = h_ref[0] + _dot(w, eo_ref[...])


def _final_body(h_ref, s_ref, b_ref, o_ref):
    hn = _ln(h_ref[0], s_ref[0], b_ref[0])
    o_ref[0] = jnp.mean(hn, 0, keepdims=True)


def _cls_body(p_ref, w_ref, b_ref, o_ref):
    o_ref[...] = _dot(p_ref[...], w_ref[...]) + b_ref[0]


def _full(shape):
    n = len(shape)
    return pl.BlockSpec(shape, lambda *_: (0,) * n)


def _per_b(shape):
    n = len(shape)
    return pl.BlockSpec((1,) + shape[1:], lambda b, *_: (b,) + (0,) * (n - 1))


def _mm3(x, w, bias):
    n, r, _ = x.shape
    return pl.pallas_call(
        _mm_body,
        grid=(n,),
        in_specs=[_per_b(x.shape), _full(w.shape), _full((1, w.shape[1]))],
        out_specs=_per_b((n, r, w.shape[1])),
        out_shape=jax.ShapeDtypeStruct((n, r, w.shape[1]), jnp.float32),
        interpret=_INTERPRET,
    )(x, w, bias.reshape(1, -1))


def _qkv_scores(hn, Wqkv, bqkv):
    return pl.pallas_call(
        _qkvs_body,
        grid=(B,),
        in_specs=[_per_b(hn.shape), _full(Wqkv.shape), _full((1, 3 * HID))],
        out_specs=[_per_b((B, S, HID)), _per_b((B, HEADS, S, S))],
        out_shape=[jax.ShapeDtypeStruct((B, S, HID), jnp.bfloat16),
                   jax.ShapeDtypeStruct((B, HEADS, S, S), jnp.float32)],
        interpret=_INTERPRET,
    )(hn, Wqkv, bqkv.reshape(1, 3 * HID))


def _attnv_proj(p, v, Wo, bo, h):
    return pl.pallas_call(
        _avproj_body,
        grid=(B,),
        in_specs=[_per_b((B, HEADS, S, S)), _per_b((B, S, HID)),
                  _full(Wo.shape), _full((1, HID)), _per_b((B, S, HID))],
        out_specs=_per_b((B, S, HID)),
        out_shape=jax.ShapeDtypeStruct((B, S, HID), jnp.float32),
        interpret=_INTERPRET,
    )(p, v, Wo, bo.reshape(1, HID), h)


def _gate(hn2, gate_w):
    return pl.pallas_call(
        _gate_body,
        out_shape=jax.ShapeDtypeStruct((T, E), jnp.float32),
        interpret=_INTERPRET,
    )(hn2, gate_w)


def _route(probs):
    return pl.pallas_call(
        _route_body,
        out_shape=[jax.ShapeDtypeStruct((T, 1), jnp.int32),
                   jax.ShapeDtypeStruct((T, 1), jnp.int32),
                   jax.ShapeDtypeStruct((T, 1), jnp.float32),
                   jax.ShapeDtypeStruct((T, 1), jnp.float32)],
        interpret=_INTERPRET,
    )(probs)


def _ffn(s1r, s2r, hn, W1, b1, W2, b2):
    return pl.pallas_call(
        _ffn_body,
        grid=(E,),
        in_specs=[_full((1, T)), _full((1, T)), _full((T, HID)),
                  pl.BlockSpec((1, HID, DFF), lambda e: (e, 0, 0)),
                  pl.BlockSpec((1, 1, DFF), lambda e: (e, 0, 0)),
                  pl.BlockSpec((1, DFF, HID), lambda e: (e, 0, 0)),
                  pl.BlockSpec((1, 1, HID), lambda e: (e, 0, 0))],
        out_specs=pl.BlockSpec((1, CAPP, HID), lambda e: (e, 0, 0)),
        out_shape=jax.ShapeDtypeStruct((E, CAPP, HID), jnp.bfloat16),
        interpret=_INTERPRET,
    )(s1r, s2r, hn, W1, b1.reshape(E, 1, DFF), W2, b2.reshape(E, 1, HID))


def _combine(s1, s2, g1, g2, h, eo):
    return pl.pallas_call(
        _comb_body,
        grid=(B,),
        in_specs=[_per_b((B, S, 1)), _per_b((B, S, 1)),
                  _per_b((B, S, 1)), _per_b((B, S, 1)),
                  _per_b((B, S, HID)), _full((E * CAPP, HID))],
        out_specs=_per_b((B, S, HID)),
        out_shape=jax.ShapeDtypeStruct((B, S, HID), jnp.float32),
        interpret=_INTERPRET,
    )(s1, s2, g1, g2, h, eo)


def _final(h, s, b):
    return pl.pallas_call(
        _final_body,
        grid=(B,),
        in_specs=[_per_b(h.shape), _full((1, HID)), _full((1, HID))],
        out_specs=_per_b((B, 1, HID)),
        out_shape=jax.ShapeDtypeStruct((B, 1, HID), jnp.float32),
        interpret=_INTERPRET,
    )(h, s.reshape(1, HID), b.reshape(1, HID))


def _classify(pooled, Wc, bc):
    return pl.pallas_call(
        _cls_body,
        out_shape=jax.ShapeDtypeStruct((B, NCLS), jnp.float32),
        interpret=_INTERPRET,
    )(pooled, Wc, bc.reshape(1, NCLS))


def kernel(x, Wpatch, bpatch, cls_tok, pos_emb, ln1_s, ln1_b, ln2_s, ln2_b,
           Wq, bq, Wk, bk, Wv, bv, Wo, bo, gate_w, W1, b1, W2, b2,
           lnf_s, lnf_b, Wc, bc):
    xp = x.reshape(B, 3, G, P, G, P).transpose(0, 2, 4, 1, 3, 5).reshape(
        B, G * G, 3 * P * P)
    pe = _mm3(xp, Wpatch.astype(jnp.bfloat16), bpatch)
    h = jnp.concatenate([jnp.broadcast_to(cls_tok, (B, 1, HID)), pe],
                        1) + pos_emb
    Wqkv = jnp.concatenate([Wq, Wk, Wv], 1).astype(jnp.bfloat16)
    bqkv = jnp.concatenate([bq, bk, bv])
    Wo_b = Wo.astype(jnp.bfloat16)
    gate_b = gate_w.astype(jnp.bfloat16)
    W1_b = W1.astype(jnp.bfloat16)
    W2_b = W2.astype(jnp.bfloat16)
    for i in range(DEPTH):
        hn = _ln(h, ln1_s[i], ln1_b[i])
        v, sc = _qkv_scores(hn, Wqkv, bqkv)
        a = jax.nn.softmax(sc, -1)
        h = _attnv_proj(a, v, Wo_b, bo, h)
        hn2 = _ln(h, ln2_s[i], ln2_b[i]).reshape(T, HID)
        probs = jax.nn.softmax(_gate(hn2, gate_b), -1)
        scat1, scat2, g1, g2 = _route(probs)
        eo = _ffn(scat1.T, scat2.T, hn2.astype(jnp.bfloat16), W1_b, b1,
                  W2_b, b2)
        h = _combine(scat1.reshape(B, S, 1), scat2.reshape(B, S, 1),
                     g1.reshape(B, S, 1), g2.reshape(B, S, 1), h,
                     eo.reshape(E * CAPP, HID))
    pooled = _final(h, lnf_s, lnf_b)
    return _classify(pooled.reshape(B, HID), Wc.astype(jnp.bfloat16), bc)


# speedup vs baseline: 1.9882x; 1.0381x over previous
import jax
import jax.numpy as jnp
from jax.experimental import pallas as pl

B = 8; IMG = 224; P = 16; HID = 768; HEADS = 12; DKV = 64; DFF = 1024
E = 16; DEPTH = 4; NCLS = 1000
G = IMG // P
S = G * G + 1
T = B * S
CAP = int(2.0 * T / E)
CAPP = 200
TRASH = E * CAPP

_INTERPRET = False


def _dot(a, b):
    return jax.lax.dot_general(a, b, (((a.ndim - 1,), (0,)), ((), ())),
                               preferred_element_type=jnp.float32)


def _ln(x, s, b):
    m = x.mean(-1, keepdims=True)
    v = ((x - m) ** 2).mean(-1, keepdims=True)
    return (x - m) / jnp.sqrt(v + 1e-6) * s + b


def _mm_body(x_ref, w_ref, b_ref, o_ref):
    o_ref[0] = _dot(x_ref[0], w_ref[...]) + b_ref[0]


def _qkvs_body(hn_ref, w_ref, b_ref, v_ref, s_ref):
    qkv = _dot(hn_ref[0], w_ref[...]) + b_ref[0]
    v_ref[0] = qkv[:, 2 * HID:].astype(jnp.bfloat16)
    for hd in range(HEADS):
        q = qkv[:, hd * DKV:(hd + 1) * DKV]
        k = qkv[:, HID + hd * DKV:HID + (hd + 1) * DKV]
        s_ref[0, hd] = jax.lax.dot_general(
            q, k, (((1,), (1,)), ((), ()))) / jnp.sqrt(jnp.float32(DKV))


def _avproj_body(p_ref, v_ref, w_ref, b_ref, h_ref, out_ref):
    os = []
    for hd in range(HEADS):
        v = v_ref[0][:, hd * DKV:(hd + 1) * DKV]
        os.append(_dot(p_ref[0, hd], v))
    o = jnp.concatenate(os, -1)
    out_ref[0] = h_ref[0] + (_dot(o, w_ref[...]) + b_ref[0])


def _route_body(hn_ref, gw_ref, scat1_ref, scat2_ref, g1_ref, g2_ref):
    probs = jax.nn.softmax(_dot(hn_ref[...], gw_ref[...]), -1)
    iota = jax.lax.broadcasted_iota(jnp.int32, (T, E), 1)

    def top(pr):
        mx = jnp.max(pr, -1, keepdims=True)
        idx = jnp.min(jnp.where(pr == mx, iota, E), -1, keepdims=True)
        m = (iota == idx).astype(jnp.float32)
        return idx, m

    idx1, m1 = top(probs)
    idx2, m2 = top(probs * (1.0 - m1))

    tri = (jax.lax.broadcasted_iota(jnp.int32, (S, S), 0)
           >= jax.lax.broadcasted_iota(jnp.int32, (S, S), 1)).astype(
               jnp.float32)

    def chunked_cumsum(m):
        off = jnp.zeros((1, E), jnp.float32)
        outs = []
        for c in range(B):
            blk = m[c * S:(c + 1) * S]
            outs.append(_dot(tri, blk) + off)
            off = off + jnp.sum(blk, 0, keepdims=True)
        return jnp.concatenate(outs, 0), off

    cs1, sum1 = chunked_cumsum(m1)
    cs2, _ = chunked_cumsum(m2)
    loc1 = cs1 - m1
    loc2 = cs2 - m2 + sum1
    m1m = m1 * (loc1 < CAP)
    m2m = m2 * (loc2 < CAP)
    p1 = jnp.sum(loc1 * m1m, -1, keepdims=True)
    p2 = jnp.sum(loc2 * m2m, -1, keepdims=True)
    g1 = jnp.sum(probs * m1m, -1, keepdims=True)
    g2 = jnp.sum(probs * m2m, -1, keepdims=True)
    den = g1 + g2 + 1e-9
    g1_ref[...] = g1 / den
    g2_ref[...] = g2 / den
    v1 = jnp.sum(m1m, -1, keepdims=True) > 0.5
    v2 = jnp.sum(m2m, -1, keepdims=True) > 0.5
    scat1_ref[...] = jnp.where(v1, idx1 * CAPP + p1.astype(jnp.int32), TRASH)
    scat2_ref[...] = jnp.where(v2, idx2 * CAPP + p2.astype(jnp.int32), TRASH)


def _moe_body(s1r_ref, s2r_ref, hn_ref, w1_ref, b1_ref, w2_ref, b2_ref,
              s1c_ref, s2c_ref, g1_ref, g2_ref, h_ref, out_ref, eo_ref):
    pid = pl.program_id(0)

    @pl.when(pid < E)
    def _expert():
        rows = jax.lax.broadcasted_iota(jnp.int32, (CAPP, T), 0) + pid * CAPP
        s1 = s1r_ref[0]
        s2 = s2r_ref[0]
        a = ((rows == s1[None, :]).astype(jnp.float32)
             + (rows == s2[None, :]).astype(jnp.float32))
        ein = _dot(a, hn_ref[...])
        h1 = jax.nn.gelu(_dot(ein, w1_ref[0]) + b1_ref[0, 0])
        eo_ref[pl.ds(pid * CAPP, CAPP), :] = _dot(h1, w2_ref[0]) + b2_ref[0, 0]

    @pl.when(pid >= E)
    def _combine_img():
        iot = jax.lax.broadcasted_iota(jnp.int32, (S, E * CAPP), 1)
        w = (g1_ref[0] * (iot == s1c_ref[0]).astype(jnp.float32)
             + g2_ref[0] * (iot == s2c_ref[0]).astype(jnp.float32))
        out_ref[0] = h_ref[0] + _dot(w, eo_ref[...])


def _final_body(h_ref, s_ref, b_ref, o_ref):
    hn = _ln(h_ref[0], s_ref[0], b_ref[0])
    o_ref[0] = jnp.mean(hn, 0, keepdims=True)


def _cls_body(p_ref, w_ref, b_ref, o_ref):
    o_ref[...] = _dot(p_ref[...], w_ref[...]) + b_ref[0]


def _full(shape):
    n = len(shape)
    return pl.BlockSpec(shape, lambda *_: (0,) * n)


def _per_b(shape):
    n = len(shape)
    return pl.BlockSpec((1,) + shape[1:], lambda b, *_: (b,) + (0,) * (n - 1))


def _mm3(x, w, bias):
    n, r, _ = x.shape
    return pl.pallas_call(
        _mm_body,
        grid=(n,),
        in_specs=[_per_b(x.shape), _full(w.shape), _full((1, w.shape[1]))],
        out_specs=_per_b((n, r, w.shape[1])),
        out_shape=jax.ShapeDtypeStruct((n, r, w.shape[1]), jnp.float32),
        interpret=_INTERPRET,
    )(x, w, bias.reshape(1, -1))


def _qkv_scores(hn, Wqkv, bqkv):
    return pl.pallas_call(
        _qkvs_body,
        grid=(B,),
        in_specs=[_per_b(hn.shape), _full(Wqkv.shape), _full((1, 3 * HID))],
        out_specs=[_per_b((B, S, HID)), _per_b((B, HEADS, S, S))],
        out_shape=[jax.ShapeDtypeStruct((B, S, HID), jnp.bfloat16),
                   jax.ShapeDtypeStruct((B, HEADS, S, S), jnp.float32)],
        interpret=_INTERPRET,
    )(hn, Wqkv, bqkv.reshape(1, 3 * HID))


def _attnv_proj(p, v, Wo, bo, h):
    return pl.pallas_call(
        _avproj_body,
        grid=(B,),
        in_specs=[_per_b((B, HEADS, S, S)), _per_b((B, S, HID)),
                  _full(Wo.shape), _full((1, HID)), _per_b((B, S, HID))],
        out_specs=_per_b((B, S, HID)),
        out_shape=jax.ShapeDtypeStruct((B, S, HID), jnp.float32),
        interpret=_INTERPRET,
    )(p, v, Wo, bo.reshape(1, HID), h)


def _route(hn2, gate_b):
    return pl.pallas_call(
        _route_body,
        out_shape=[jax.ShapeDtypeStruct((T, 1), jnp.int32),
                   jax.ShapeDtypeStruct((T, 1), jnp.int32),
                   jax.ShapeDtypeStruct((T, 1), jnp.float32),
                   jax.ShapeDtypeStruct((T, 1), jnp.float32)],
        interpret=_INTERPRET,
    )(hn2, gate_b)


def _moe(s1r, s2r, hn, W1, b1, W2, b2, s1, s2, g1, g2, h):
    from jax.experimental.pallas import tpu as pltpu
    ew = lambda e: (jnp.minimum(e, E - 1), 0, 0)
    img = lambda e: (jnp.maximum(e - E, 0), 0, 0)
    return pl.pallas_call(
        _moe_body,
        grid=(E + B,),
        in_specs=[_full((1, T)), _full((1, T)), _full((T, HID)),
                  pl.BlockSpec((1, HID, DFF), ew),
                  pl.BlockSpec((1, 1, DFF), ew),
                  pl.BlockSpec((1, DFF, HID), ew),
                  pl.BlockSpec((1, 1, HID), ew),
                  pl.BlockSpec((1, S, 1), img), pl.BlockSpec((1, S, 1), img),
                  pl.BlockSpec((1, S, 1), img), pl.BlockSpec((1, S, 1), img),
                  pl.BlockSpec((1, S, HID), img)],
        out_specs=pl.BlockSpec((1, S, HID), img),
        out_shape=jax.ShapeDtypeStruct((B, S, HID), jnp.float32),
        scratch_shapes=[pltpu.VMEM((E * CAPP, HID), jnp.float32)],
        interpret=_INTERPRET,
    )(s1r, s2r, hn, W1, b1.reshape(E, 1, DFF), W2, b2.reshape(E, 1, HID),
      s1, s2, g1, g2, h)


def _final(h, s, b):
    return pl.pallas_call(
        _final_body,
        grid=(B,),
        in_specs=[_per_b(h.shape), _full((1, HID)), _full((1, HID))],
        out_specs=_per_b((B, 1, HID)),
        out_shape=jax.ShapeDtypeStruct((B, 1, HID), jnp.float32),
        interpret=_INTERPRET,
    )(h, s.reshape(1, HID), b.reshape(1, HID))


def _classify(pooled, Wc, bc):
    return pl.pallas_call(
        _cls_body,
        out_shape=jax.ShapeDtypeStruct((B, NCLS), jnp.float32),
        interpret=_INTERPRET,
    )(pooled, Wc, bc.reshape(1, NCLS))


def kernel(x, Wpatch, bpatch, cls_tok, pos_emb, ln1_s, ln1_b, ln2_s, ln2_b,
           Wq, bq, Wk, bk, Wv, bv, Wo, bo, gate_w, W1, b1, W2, b2,
           lnf_s, lnf_b, Wc, bc):
    xp = x.reshape(B, 3, G, P, G, P).transpose(0, 2, 4, 1, 3, 5).reshape(
        B, G * G, 3 * P * P)
    pe = _mm3(xp, Wpatch.astype(jnp.bfloat16), bpatch)
    h = jnp.concatenate([jnp.broadcast_to(cls_tok, (B, 1, HID)), pe],
                        1) + pos_emb
    Wqkv = jnp.concatenate([Wq, Wk, Wv], 1).astype(jnp.bfloat16)
    bqkv = jnp.concatenate([bq, bk, bv])
    Wo_b = Wo.astype(jnp.bfloat16)
    gate_b = gate_w.astype(jnp.bfloat16)
    W1_b = W1.astype(jnp.bfloat16)
    W2_b = W2.astype(jnp.bfloat16)
    for i in range(DEPTH):
        hn = _ln(h, ln1_s[i], ln1_b[i])
        v, sc = _qkv_scores(hn, Wqkv, bqkv)
        a = jax.nn.softmax(sc, -1)
        h = _attnv_proj(a, v, Wo_b, bo, h)
        hn2 = _ln(h, ln2_s[i], ln2_b[i]).reshape(T, HID)
        scat1, scat2, g1, g2 = _route(hn2, gate_b)
        h = _moe(scat1.T, scat2.T, hn2.astype(jnp.bfloat16), W1_b, b1,
                 W2_b, b2, scat1.reshape(B, S, 1), scat2.reshape(B, S, 1),
                 g1.reshape(B, S, 1), g2.reshape(B, S, 1), h)
    pooled = _final(h, lnf_s, lnf_b)
    return _classify(pooled.reshape(B, HID), Wc.astype(jnp.bfloat16), bc)
